# baseline (speedup 1.0000x reference)
import functools

import jax
import jax.numpy as jnp
from jax import lax
from jax.experimental import pallas as pl
from jax.experimental.pallas import tpu as pltpu
from jax.experimental.pallas import tpu_sc as plsc

N_NODES = 10000
N_EDGES = 160000
TEXT_REP = 256
NODE_TYPE_EMB = 128
EDGE_TYPE_EMB = 16
NODE_HIDDEN = 256
NUM_NODE_TYPES = 16

_NC = 2
_NS = 16
_NW = _NC * _NS
_EPW = N_EDGES // _NW
_EPWP = _EPW + 16 - (_EPW % 16)
_NG = (_EPW + 15) // 16

_RB = 2000
_G = N_NODES // _RB


def _node_body(ids_ref, text_ref, ntt_ref, w_ref, b_ref, out_ref):
    ft = jnp.dot(ntt_ref[:], w_ref[:NODE_TYPE_EMB, :],
                 preferred_element_type=jnp.float32) + b_ref[:]
    ids = ids_ref[0, 0, :]
    onehot = (ids[:, None] == lax.broadcasted_iota(
        jnp.int32, (_RB, NUM_NODE_TYPES), 1)).astype(jnp.float32)
    acc = jnp.dot(text_ref[:], w_ref[NODE_TYPE_EMB:, :],
                  preferred_element_type=jnp.float32)
    out_ref[:] = acc + jnp.dot(onehot, ft,
                               preferred_element_type=jnp.float32)


def _node_proj(ids3, text, ntt, w, b2):
    return pl.pallas_call(
        _node_body,
        grid=(_G,),
        in_specs=[
            pl.BlockSpec((1, 1, _RB), lambda i: (i, 0, 0)),
            pl.BlockSpec((_RB, TEXT_REP), lambda i: (i, 0)),
            pl.BlockSpec((NUM_NODE_TYPES, NODE_TYPE_EMB), lambda i: (0, 0)),
            pl.BlockSpec((NODE_TYPE_EMB + TEXT_REP, NODE_HIDDEN),
                         lambda i: (0, 0)),
            pl.BlockSpec((1, NODE_HIDDEN), lambda i: (0, 0)),
        ],
        out_specs=pl.BlockSpec((_RB, NODE_HIDDEN), lambda i: (i, 0)),
        out_shape=jax.ShapeDtypeStruct((N_NODES, NODE_HIDDEN), jnp.float32),
    )(ids3, text, ntt, w, b2)


_CH = 400
_NFC = _EPW // _CH
_TAIL = _EPW - _NFC * _CH


def _edge_gather(table, ids):
    mesh = plsc.VectorSubcoreMesh(core_axis_name="c", subcore_axis_name="s")

    @functools.partial(
        pl.kernel, mesh=mesh,
        compiler_params=pltpu.CompilerParams(
            needs_layout_passes=False, use_tc_tiling_on_sc=True),
        out_type=jax.ShapeDtypeStruct((N_EDGES, EDGE_TYPE_EMB), jnp.float32),
        scratch_types=[
            pltpu.VMEM((8, EDGE_TYPE_EMB), jnp.float32),
            pltpu.VMEM((_EPWP,), jnp.int32),
            pltpu.VMEM((_CH + 16, EDGE_TYPE_EMB), jnp.float32),
        ],
    )
    def k(table_hbm, idx_hbm, out_hbm, table_v, idx_v, out_c):
        wid = lax.axis_index("s") * _NC + lax.axis_index("c")
        lane = lax.broadcasted_iota(jnp.int32, (16,), 0)
        idx_v[pl.ds(_EPWP - 16, 16)] = jnp.zeros((16,), jnp.int32)
        pltpu.sync_copy(table_hbm, table_v)
        pltpu.sync_copy(idx_hbm.at[pl.ds(wid * _EPW, _EPW)], idx_v.at[pl.ds(0, _EPW)])
        cols = [jnp.full((16,), d, jnp.int32) for d in range(EDGE_TYPE_EMB)]

        def group(base, g, rows):
            ids16 = idx_v[pl.ds(base + g * 16, 16)]
            vals = [plsc.load_gather(table_v, [ids16, cols[d]])
                    for d in range(EDGE_TYPE_EMB)]
            for d in range(EDGE_TYPE_EMB):
                plsc.store_scatter(out_c, [rows, cols[d]], vals[d])

        def chunk_body(c, carry):
            base = c * _CH
            for g in range(_CH // 16):
                group(base, g, g * 16 + lane)
            pltpu.sync_copy(out_c.at[pl.ds(0, _CH)],
                            out_hbm.at[pl.ds(wid * _EPW + base, _CH)])
            return carry

        lax.fori_loop(0, _NFC, chunk_body, 0)
        tbase = _NFC * _CH
        for g in range(_TAIL // 16 + 1):
            group(tbase, g, g * 16 + lane)
        pltpu.sync_copy(out_c.at[pl.ds(0, _TAIL)],
                        out_hbm.at[pl.ds(wid * _EPW + tbase, _TAIL)])

    return k(table, ids)


def kernel(node_type_ids, text_encodings, edge_index, edge_type_ids,
           node_type_table, edge_type_table, W, b):
    del edge_index
    ids3 = node_type_ids.astype(jnp.int32).reshape(_G, 1, _RB)
    b2 = b.reshape(1, NODE_HIDDEN)
    node_h = _node_proj(ids3, text_encodings, node_type_table, W, b2)
    edge_h = _edge_gather(edge_type_table, edge_type_ids.astype(jnp.int32))
    return node_h, edge_h


# speedup vs baseline: 5.0403x; 5.0403x over previous
import functools

import jax
import jax.numpy as jnp
from jax import lax
from jax.experimental import pallas as pl
from jax.experimental.pallas import tpu as pltpu
from jax.experimental.pallas import tpu_sc as plsc

N_NODES = 10000
N_EDGES = 160000
TEXT_REP = 256
NODE_TYPE_EMB = 128
EDGE_TYPE_EMB = 16
NODE_HIDDEN = 256
NUM_NODE_TYPES = 16

_NC = 2
_NS = 16
_NW = _NC * _NS
_BW = 4992
_BL = N_EDGES - _BW * (_NW - 1)

_RB = 5000
_G = N_NODES // _RB


def _node_body(ids_ref, text_ref, ntt_ref, w_ref, b_ref, out_ref):
    ft = jnp.dot(ntt_ref[:], w_ref[:NODE_TYPE_EMB, :],
                 preferred_element_type=jnp.float32) + b_ref[:]
    ids = ids_ref[0, 0, :]
    onehot = (ids[:, None] == lax.broadcasted_iota(
        jnp.int32, (_RB, NUM_NODE_TYPES), 1)).astype(jnp.float32)
    acc = jnp.dot(text_ref[:], w_ref[NODE_TYPE_EMB:, :],
                  preferred_element_type=jnp.float32)
    out_ref[:] = acc + jnp.dot(onehot, ft,
                               preferred_element_type=jnp.float32)


def _node_proj(ids3, text, ntt, w, b2):
    return pl.pallas_call(
        _node_body,
        grid=(_G,),
        in_specs=[
            pl.BlockSpec((1, 1, _RB), lambda i: (i, 0, 0)),
            pl.BlockSpec((_RB, TEXT_REP), lambda i: (i, 0)),
            pl.BlockSpec((NUM_NODE_TYPES, NODE_TYPE_EMB), lambda i: (0, 0)),
            pl.BlockSpec((NODE_TYPE_EMB + TEXT_REP, NODE_HIDDEN),
                         lambda i: (0, 0)),
            pl.BlockSpec((1, NODE_HIDDEN), lambda i: (0, 0)),
        ],
        out_specs=pl.BlockSpec((_RB, NODE_HIDDEN), lambda i: (i, 0)),
        out_shape=jax.ShapeDtypeStruct((N_NODES, NODE_HIDDEN), jnp.float32),
    )(ids3, text, ntt, w, b2)


def _edge_gather_t(table, ids):
    mesh = plsc.VectorSubcoreMesh(core_axis_name="c", subcore_axis_name="s")

    @functools.partial(
        pl.kernel, mesh=mesh,
        compiler_params=pltpu.CompilerParams(
            needs_layout_passes=False, use_tc_tiling_on_sc=True),
        out_type=jax.ShapeDtypeStruct((EDGE_TYPE_EMB, N_EDGES), jnp.float32),
        scratch_types=[
            pltpu.VMEM((16, 16), jnp.float32),
            pltpu.VMEM((_BL,), jnp.int32),
            pltpu.VMEM((EDGE_TYPE_EMB, _BL), jnp.float32),
            pltpu.SemaphoreType.DMA,
        ],
    )
    def k(table_hbm, idx_hbm, out_hbm, table_v, idx_v, out_c, sem):
        wid = lax.axis_index("s") * _NC + lax.axis_index("c")
        base = wid * _BW
        is_last = wid == _NW - 1
        pltpu.sync_copy(table_hbm, table_v)
        pltpu.sync_copy(idx_hbm.at[pl.ds(base, _BW)], idx_v.at[pl.ds(0, _BW)])

        @pl.when(is_last)
        def _():
            pltpu.sync_copy(idx_hbm.at[pl.ds(base + _BW, _BL - _BW)],
                            idx_v.at[pl.ds(_BW, _BL - _BW)])

        tcols = [table_v[d, :] for d in range(EDGE_TYPE_EMB)]
        dnums = lax.GatherDimensionNumbers(
            offset_dims=(), collapsed_slice_dims=(0,), start_index_map=(0,))

        def reg_take(col, idx):
            return lax.gather(col, idx[:, None], dnums, (1,),
                              mode=lax.GatherScatterMode.PROMISE_IN_BOUNDS)

        def tile_col(cbase):
            for gi in range(8):
                ids16 = idx_v[pl.ds(cbase + gi * 16, 16)]
                vals = [reg_take(tcols[d], ids16)
                        for d in range(EDGE_TYPE_EMB)]
                for d in range(EDGE_TYPE_EMB):
                    out_c[d, pl.ds(cbase + gi * 16, 16)] = vals[d]

        ntc = jnp.where(is_last, _BL // 128, _BW // 128)

        def body(c, carry):
            cb = c * 128
            tile_col(cb)
            pltpu.async_copy(out_c.at[:, pl.ds(cb, 128)],
                             out_hbm.at[:, pl.ds(base + cb, 128)], sem)

            @pl.when(c > 0)
            def _():
                pltpu.make_async_copy(
                    out_c.at[:, pl.ds(0, 128)],
                    out_hbm.at[:, pl.ds(base, 128)], sem).wait()

            return carry

        lax.fori_loop(0, ntc, body, 0)
        pltpu.make_async_copy(out_c.at[:, pl.ds(0, 128)],
                              out_hbm.at[:, pl.ds(base, 128)], sem).wait()

    return k(table, ids)


def kernel(node_type_ids, text_encodings, edge_index, edge_type_ids,
           node_type_table, edge_type_table, W, b):
    del edge_index
    ids3 = node_type_ids.astype(jnp.int32).reshape(_G, 1, _RB)
    b2 = b.reshape(1, NODE_HIDDEN)
    node_h = _node_proj(ids3, text_encodings, node_type_table, W, b2)
    tt = jnp.pad(edge_type_table.T, ((0, 0), (0, 8)))
    edge_h = _edge_gather_t(tt, edge_type_ids.astype(jnp.int32)).T
    return node_h, edge_h


# speedup vs baseline: 5.2069x; 1.0331x over previous
import functools

import jax
import jax.numpy as jnp
from jax import lax
from jax.experimental import pallas as pl
from jax.experimental.pallas import tpu as pltpu
from jax.experimental.pallas import tpu_sc as plsc

N_NODES = 10000
N_EDGES = 160000
TEXT_REP = 256
NODE_TYPE_EMB = 128
EDGE_TYPE_EMB = 16
NODE_HIDDEN = 256
NUM_NODE_TYPES = 16

_NC = 2
_NS = 16
_NW = _NC * _NS
_BW = 4992
_BL = N_EDGES - _BW * (_NW - 1)

_RB = 5000
_G = N_NODES // _RB


def _node_body(ids_ref, text_ref, ntt_ref, w_ref, b_ref, out_ref):
    ft = jnp.dot(ntt_ref[:], w_ref[:NODE_TYPE_EMB, :],
                 preferred_element_type=jnp.float32) + b_ref[:]
    ids = ids_ref[0, 0, :]
    onehot = (ids[:, None] == lax.broadcasted_iota(
        jnp.int32, (_RB, NUM_NODE_TYPES), 1)).astype(jnp.float32)
    acc = jnp.dot(text_ref[:], w_ref[NODE_TYPE_EMB:, :],
                  preferred_element_type=jnp.float32)
    out_ref[:] = acc + jnp.dot(onehot, ft,
                               preferred_element_type=jnp.float32)


def _node_proj(ids3, text, ntt, w, b2):
    return pl.pallas_call(
        _node_body,
        grid=(_G,),
        compiler_params=pltpu.CompilerParams(
            allow_input_fusion=[True, True, True, True, True]),
        in_specs=[
            pl.BlockSpec((1, 1, _RB), lambda i: (i, 0, 0)),
            pl.BlockSpec((_RB, TEXT_REP), lambda i: (i, 0)),
            pl.BlockSpec((NUM_NODE_TYPES, NODE_TYPE_EMB), lambda i: (0, 0)),
            pl.BlockSpec((NODE_TYPE_EMB + TEXT_REP, NODE_HIDDEN),
                         lambda i: (0, 0)),
            pl.BlockSpec((1, NODE_HIDDEN), lambda i: (0, 0)),
        ],
        out_specs=pl.BlockSpec((_RB, NODE_HIDDEN), lambda i: (i, 0)),
        out_shape=jax.ShapeDtypeStruct((N_NODES, NODE_HIDDEN), jnp.float32),
    )(ids3, text, ntt, w, b2)


def _edge_gather_t(table, ids):
    mesh = plsc.VectorSubcoreMesh(core_axis_name="c", subcore_axis_name="s")

    @functools.partial(
        pl.kernel, mesh=mesh,
        compiler_params=pltpu.CompilerParams(
            needs_layout_passes=False, use_tc_tiling_on_sc=True),
        out_type=jax.ShapeDtypeStruct((EDGE_TYPE_EMB, N_EDGES), jnp.float32),
        scratch_types=[
            pltpu.VMEM((8, EDGE_TYPE_EMB), jnp.float32),
            pltpu.VMEM((_BL,), jnp.int32),
            pltpu.VMEM((EDGE_TYPE_EMB, _BL), jnp.float32),
            pltpu.SemaphoreType.DMA,
        ],
    )
    def k(table_hbm, idx_hbm, out_hbm, table_v, idx_v, out_c, sem):
        wid = lax.axis_index("s") * _NC + lax.axis_index("c")
        base = wid * _BW
        is_last = wid == _NW - 1
        pltpu.sync_copy(table_hbm, table_v)
        pltpu.sync_copy(idx_hbm.at[pl.ds(base, _BW)], idx_v.at[pl.ds(0, _BW)])

        @pl.when(is_last)
        def _():
            pltpu.sync_copy(idx_hbm.at[pl.ds(base + _BW, _BL - _BW)],
                            idx_v.at[pl.ds(_BW, _BL - _BW)])

        dnums = lax.GatherDimensionNumbers(
            offset_dims=(), collapsed_slice_dims=(0,), start_index_map=(0,))

        def reg_take(col, idx):
            return lax.gather(col, idx[:, None], dnums, (1,),
                              mode=lax.GatherScatterMode.PROMISE_IN_BOUNDS)

        lane = lax.broadcasted_iota(jnp.int32, (16,), 0)
        trows = [table_v[i, :] for i in range(8)]
        tcols = []
        for d in range(EDGE_TYPE_EMB):
            dvec = jnp.full((16,), d, jnp.int32)
            col = reg_take(trows[0], dvec)
            for i in range(1, 8):
                col = jnp.where(lane == i, reg_take(trows[i], dvec), col)
            tcols.append(col)

        def tile_col(cbase):
            for gi in range(8):
                ids16 = idx_v[pl.ds(cbase + gi * 16, 16)]
                vals = [reg_take(tcols[d], ids16)
                        for d in range(EDGE_TYPE_EMB)]
                for d in range(EDGE_TYPE_EMB):
                    out_c[d, pl.ds(cbase + gi * 16, 16)] = vals[d]

        ntc = jnp.where(is_last, _BL // 128, _BW // 128)

        def body(c, carry):
            cb = c * 128
            tile_col(cb)
            pltpu.async_copy(out_c.at[:, pl.ds(cb, 128)],
                             out_hbm.at[:, pl.ds(base + cb, 128)], sem)

            @pl.when(c > 0)
            def _():
                pltpu.make_async_copy(
                    out_c.at[:, pl.ds(0, 128)],
                    out_hbm.at[:, pl.ds(base, 128)], sem).wait()

            return carry

        lax.fori_loop(0, ntc, body, 0)
        pltpu.make_async_copy(out_c.at[:, pl.ds(0, 128)],
                              out_hbm.at[:, pl.ds(base, 128)], sem).wait()

    return k(table, ids)


def kernel(node_type_ids, text_encodings, edge_index, edge_type_ids,
           node_type_table, edge_type_table, W, b):
    del edge_index
    ids3 = node_type_ids.astype(jnp.int32).reshape(_G, 1, _RB)
    b2 = b.reshape(1, NODE_HIDDEN)
    node_h = _node_proj(ids3, text_encodings, node_type_table, W, b2)
    edge_h = _edge_gather_t(edge_type_table,
                            edge_type_ids.astype(jnp.int32)).T
    return node_h, edge_h
